# baseline (speedup 1.0000x reference)
import jax
import jax.numpy as jnp
from jax.experimental import pallas as pl
from jax.experimental.pallas import tpu as pltpu

NMS_KSIZE = 5
TOP_K = 512
GAUSS_KSIZE = 5
GAUSS_SIGMA = 0.5
LOSS_LAMBDA = 100.0


def _filter_border(x, radius=8):
    m = jnp.zeros_like(x)
    m = m.at[:, :, radius:-radius, radius:-radius].set(1.0)
    return x * m


def _bilinear_sample(img, x, y):
    B, C, H, W = img.shape
    x0 = jnp.floor(x); x1 = x0 + 1.0
    y0 = jnp.floor(y); y1 = y0 + 1.0
    wa = (x1 - x) * (y1 - y)
    wb = (x1 - x) * (y - y0)
    wc = (x - x0) * (y1 - y)
    wd = (x - x0) * (y - y0)
    flat = img.reshape(B, C, H * W)

    def gather(xi, yi):
        valid = ((xi >= 0) & (xi <= W - 1) & (yi >= 0) & (yi <= H - 1)).astype(img.dtype)
        xc = jnp.clip(xi, 0, W - 1).astype(jnp.int32)
        yc = jnp.clip(yi, 0, H - 1).astype(jnp.int32)
        idx = yc * W + xc
        vals = jnp.take_along_axis(flat, idx[:, None, :], axis=2)
        return vals * valid[:, None, :]

    out = (gather(x0, y0) * wa[:, None, :] + gather(x0, y1) * wb[:, None, :]
           + gather(x1, y0) * wc[:, None, :] + gather(x1, y1) * wd[:, None, :])
    return out.reshape(B, C, H, W)


def _warp_image(img, homo):
    B, C, H, W = img.shape
    ys, xs = jnp.meshgrid(jnp.arange(H, dtype=jnp.float32), jnp.arange(W, dtype=jnp.float32), indexing='ij')
    grid = jnp.stack([xs.reshape(-1), ys.reshape(-1), jnp.ones(H * W, dtype=jnp.float32)], axis=0)
    warped = homo @ grid[None]
    warped = warped / (warped[:, 2:3, :] + 1e-8)
    return _bilinear_sample(img, warped[:, 0, :], warped[:, 1, :])


def _select_keypoints(score, nms_ksize, top_k):
    B, C, H, W = score.shape
    pad = nms_ksize // 2
    maxed = jax.lax.reduce_window(score, -jnp.inf, jax.lax.max,
                                  (1, 1, nms_ksize, nms_ksize), (1, 1, 1, 1),
                                  [(0, 0), (0, 0), (pad, pad), (pad, pad)])
    mask = ((score == maxed) & (score >= 0.0)).astype(score.dtype)
    nms_score = score * mask
    flat = nms_score.reshape(B, H * W)
    vals, idx = jax.lax.top_k(flat, top_k)
    topk_mask = jnp.zeros_like(flat).at[jnp.arange(B)[:, None], idx].set(1.0)
    out = (flat * topk_mask).reshape(B, C, H, W)
    kp = jnp.stack([idx % W, idx // W], axis=-1)
    return out, kp


def _gaussian_filter(x, ksize, sigma):
    ax = jnp.arange(ksize, dtype=jnp.float32) - (ksize - 1) / 2.0
    g = jnp.exp(-(ax ** 2) / (2.0 * sigma ** 2))
    k2 = jnp.outer(g, g)
    k2 = k2 / k2.sum()
    kern = k2[None, None, :, :]
    pad = ksize // 2
    return jax.lax.conv_general_dilated(x, kern, (1, 1), [(pad, pad), (pad, pad)],
                                        dimension_numbers=('NCHW', 'OIHW', 'NCHW'))


def _loss_body(s1_ref, gt_ref, vis_ref, sq_ref, vsum_ref):
    b = pl.program_id(0)
    s1 = s1_ref[...]
    gt = gt_ref[...]
    vis = vis_ref[...]
    d = s1 - gt
    sq = jnp.sum(d * d * vis)
    vs = jnp.sum(vis)
    rows = jax.lax.broadcasted_iota(jnp.int32, sq_ref.shape, 0)
    sel = (rows == b).astype(jnp.float32)

    @pl.when(b == 0)
    def _():
        sq_ref[...] = jnp.zeros_like(sq_ref)
        vsum_ref[...] = jnp.zeros_like(vsum_ref)

    sq_ref[...] += sel * sq
    vsum_ref[...] += sel * vs


def kernel(score1, score2, homo12):
    B, C, H, W = score1.shape
    score2b = _filter_border(score2)
    w_score2 = _warp_image(score2b, homo12)
    vis_mask = jnp.ones_like(score2)
    vis_mask1 = (_warp_image(vis_mask, homo12) > 0).astype(jnp.float32)
    s1, kp1 = _select_keypoints(score1, NMS_KSIZE, TOP_K)
    s1 = _gaussian_filter(s1, GAUSS_KSIZE, GAUSS_SIGMA)
    gt_s1, _ = _select_keypoints(w_score2, NMS_KSIZE, TOP_K)
    gt_s1 = _gaussian_filter(gt_s1, GAUSS_KSIZE, GAUSS_SIGMA)

    sq, vsum = pl.pallas_call(
        _loss_body,
        grid=(B,),
        in_specs=[
            pl.BlockSpec((1, 1, H, W), lambda b: (b, 0, 0, 0)),
            pl.BlockSpec((1, 1, H, W), lambda b: (b, 0, 0, 0)),
            pl.BlockSpec((1, 1, H, W), lambda b: (b, 0, 0, 0)),
        ],
        out_specs=[
            pl.BlockSpec((B, 128), lambda b: (0, 0)),
            pl.BlockSpec((B, 128), lambda b: (0, 0)),
        ],
        out_shape=[
            jax.ShapeDtypeStruct((B, 128), jnp.float32),
            jax.ShapeDtypeStruct((B, 128), jnp.float32),
        ],
    )(s1, gt_s1, vis_mask1)
    sq = sq[:, :1]
    vsum = vsum[:, :1]

    loss = sq.sum() / vsum.sum() * LOSS_LAMBDA
    return (loss, kp1, vis_mask1)


# speedup vs baseline: 6.7143x; 6.7143x over previous
import functools

import jax
import jax.numpy as jnp
from jax import lax
from jax.experimental import pallas as pl
from jax.experimental.pallas import tpu as pltpu
from jax.experimental.pallas import tpu_sc as plsc

B, H, W = 16, 512, 512
HW = H * W
TOP_K = 512
CAP = 1024
BAND_ROWS = 86
RES_ROWS = 88
NBANDS = 6
CHUNK = 8192
LOSS_LAMBDA = 100.0

_G0 = 1.0
_G1 = 0.1353352832366127
_G2 = 0.00033546262790251185
_GSUM = _G0 + 2.0 * _G1 + 2.0 * _G2
_INVZ = 1.0 / (_GSUM * _GSUM)


def _warp_prep_body(xw_ref, yw_ref, s1_ref, planes_ref, fw_ref, vis_ref,
                    nms1_ref, t1_ref, bounds_ref):
    b = pl.program_id(0)
    xw = xw_ref[0, 0]
    yw = yw_ref[0, 0]

    x0 = jnp.floor(xw)
    y0 = jnp.floor(yw)
    x1 = x0 + 1.0
    y1 = y0 + 1.0
    fx = xw - x0
    fy = yw - y0
    wa = (1.0 - fx) * (1.0 - fy)
    wb = (1.0 - fx) * fy
    wc = fx * (1.0 - fy)
    wd = fx * fy

    def corner(xi, yi, wgt):
        valid = ((xi >= 0.0) & (xi <= W - 1.0) & (yi >= 0.0) & (yi <= H - 1.0))
        xc = jnp.clip(xi, 0.0, W - 1.0)
        yc = jnp.clip(yi, 0.0, H - 1.0)
        inb = ((xc >= 8.0) & (xc <= W - 9.0) & (yc >= 8.0) & (yc <= H - 9.0))
        fw = wgt * valid.astype(jnp.float32) * inb.astype(jnp.float32)
        idx = yc.astype(jnp.int32) * W + xc.astype(jnp.int32)
        vw = wgt * valid.astype(jnp.float32)
        return idx, fw, vw

    ia, fwa, va = corner(x0, y0, wa)
    ib, fwb, vb = corner(x0, y1, wb)
    ic, fwc, vc = corner(x1, y0, wc)
    idd, fwd, vd = corner(x1, y1, wd)

    vis = ((va + vb + vc + vd) > 0.0).astype(jnp.float32)
    vis_ref[...] = vis[None, None]

    planes_ref[...] = jnp.stack([ia, ib, ic, idd])[None]
    fw_ref[...] = jnp.stack([fwa, fwb, fwc, fwd])[None]

    rowa = (ia >> 9).astype(jnp.float32).reshape(32, 16 * W)
    rowd = (idd >> 9).astype(jnp.float32).reshape(32, 16 * W)
    rmin = jnp.min(rowa, axis=1, keepdims=True)
    rmax = jnp.max(rowd, axis=1, keepdims=True)
    bounds = jnp.concatenate([jnp.broadcast_to(rmin, (32, 128)),
                              jnp.broadcast_to(rmax, (32, 128))], axis=0)
    bounds_ref[...] = bounds[None]

    s = s1_ref[0, 0]
    zc = jnp.zeros((H, 2), jnp.float32)
    p = jnp.concatenate([zc, s, zc], axis=1)
    mh = p[:, 0:W]
    for k in range(1, 5):
        mh = jnp.maximum(mh, p[:, k:k + W])
    zr = jnp.zeros((2, W), jnp.float32)
    q = jnp.concatenate([zr, mh, zr], axis=0)
    mx = q[0:H, :]
    for k in range(1, 5):
        mx = jnp.maximum(mx, q[k:k + H, :])
    nms = s * ((s == mx) & (s >= 0.0)).astype(jnp.float32)
    nms1_ref[...] = nms[None, None]

    m = nms
    for _ in range(5):
        wth = m.shape[1] // 2
        m = jnp.maximum(m[:, :wth], m[:, wth:])
    bits = lax.bitcast_convert_type(m, jnp.int32)

    def bs_body(_, carry):
        lo, hi = carry
        mid = lo + (hi - lo) // 2
        cnt = jnp.sum((bits >= mid).astype(jnp.float32))
        big = cnt >= float(TOP_K)
        return (jnp.where(big, mid, lo), jnp.where(big, hi, mid))

    lo, _ = lax.fori_loop(0, 31, bs_body, (jnp.int32(0), jnp.int32(0x7F800001)))
    t = lax.bitcast_convert_type(lo, jnp.float32)

    @pl.when(b == 0)
    def _():
        t1_ref[...] = jnp.zeros_like(t1_ref)

    rows = lax.broadcasted_iota(jnp.int32, t1_ref.shape, 0)
    t1_ref[...] += jnp.where(rows == b, t, 0.0)


def _warp_prep(xw, yw, score1):
    return pl.pallas_call(
        _warp_prep_body,
        grid=(B,),
        in_specs=[
            pl.BlockSpec((1, 1, H, W), lambda b: (b, 0, 0, 0)),
            pl.BlockSpec((1, 1, H, W), lambda b: (b, 0, 0, 0)),
            pl.BlockSpec((1, 1, H, W), lambda b: (b, 0, 0, 0)),
        ],
        out_specs=[
            pl.BlockSpec((1, 4, H, W), lambda b: (b, 0, 0, 0)),
            pl.BlockSpec((1, 4, H, W), lambda b: (b, 0, 0, 0)),
            pl.BlockSpec((1, 1, H, W), lambda b: (b, 0, 0, 0)),
            pl.BlockSpec((1, 1, H, W), lambda b: (b, 0, 0, 0)),
            pl.BlockSpec((B, 128), lambda b: (0, 0)),
            pl.BlockSpec((1, 64, 128), lambda b: (b, 0, 0)),
        ],
        out_shape=[
            jax.ShapeDtypeStruct((B, 4, H, W), jnp.int32),
            jax.ShapeDtypeStruct((B, 4, H, W), jnp.float32),
            jax.ShapeDtypeStruct((B, 1, H, W), jnp.float32),
            jax.ShapeDtypeStruct((B, 1, H, W), jnp.float32),
            jax.ShapeDtypeStruct((B, 128), jnp.float32),
            jax.ShapeDtypeStruct((B, 64, 128), jnp.float32),
        ],
    )(xw, yw, score1)


def _nms2_body(w_ref, nms2_ref, t2_ref):
    b = pl.program_id(0)
    s = w_ref[0, 0]
    zc = jnp.zeros((H, 2), jnp.float32)
    p = jnp.concatenate([zc, s, zc], axis=1)
    mh = p[:, 0:W]
    for k in range(1, 5):
        mh = jnp.maximum(mh, p[:, k:k + W])
    zr = jnp.zeros((2, W), jnp.float32)
    q = jnp.concatenate([zr, mh, zr], axis=0)
    mx = q[0:H, :]
    for k in range(1, 5):
        mx = jnp.maximum(mx, q[k:k + H, :])
    nms = s * ((s == mx) & (s >= 0.0)).astype(jnp.float32)
    nms2_ref[...] = nms[None, None]

    m = nms
    for _ in range(5):
        wth = m.shape[1] // 2
        m = jnp.maximum(m[:, :wth], m[:, wth:])
    bits = lax.bitcast_convert_type(m, jnp.int32)

    def bs_body(_, carry):
        lo, hi = carry
        mid = lo + (hi - lo) // 2
        cnt = jnp.sum((bits >= mid).astype(jnp.float32))
        big = cnt >= float(TOP_K)
        return (jnp.where(big, mid, lo), jnp.where(big, hi, mid))

    lo, _ = lax.fori_loop(0, 31, bs_body, (jnp.int32(0), jnp.int32(0x7F800001)))
    t = lax.bitcast_convert_type(lo, jnp.float32)

    @pl.when(b == 0)
    def _():
        t2_ref[...] = jnp.zeros_like(t2_ref)

    rows = lax.broadcasted_iota(jnp.int32, t2_ref.shape, 0)
    t2_ref[...] += jnp.where(rows == b, t, 0.0)


def _nms2(w_score2):
    return pl.pallas_call(
        _nms2_body,
        grid=(B,),
        in_specs=[pl.BlockSpec((1, 1, H, W), lambda b: (b, 0, 0, 0))],
        out_specs=[
            pl.BlockSpec((1, 1, H, W), lambda b: (b, 0, 0, 0)),
            pl.BlockSpec((B, 128), lambda b: (0, 0)),
        ],
        out_shape=[
            jax.ShapeDtypeStruct((B, 1, H, W), jnp.float32),
            jax.ShapeDtypeStruct((B, 128), jnp.float32),
        ],
    )(w_score2)


def _sc_gather_kernel():
    mesh = plsc.VectorSubcoreMesh(core_axis_name="c", subcore_axis_name="s")
    info = plsc.get_sparse_core_info()
    nc = info.num_cores

    @functools.partial(
        pl.kernel,
        mesh=mesh,
        out_type=jax.ShapeDtypeStruct((B, HW), jnp.float32),
        compiler_params=pltpu.CompilerParams(needs_layout_passes=False),
        scratch_types=[
            pltpu.VMEM((4, 16, W), jnp.int32),
            pltpu.VMEM((4, 16, W), jnp.float32),
            pltpu.VMEM((RES_ROWS * W // 128, 128), jnp.float32),
            pltpu.VMEM((CHUNK,), jnp.float32),
            pltpu.VMEM((64, 128), jnp.float32),
        ],
    )
    def k(planes_hbm, fwp_hbm, s2_hbm, bounds_hbm, out_hbm, stg, stgf, band,
          acc, bv):
        wid = lax.axis_index("s") * nc + lax.axis_index("c")
        b = wid // 2
        hh = wid % 2
        pltpu.sync_copy(bounds_hbm.at[b], bv)

        def do_chunk(ch, _):
            r0 = hh * 256 + ch * 16
            pltpu.sync_copy(planes_hbm.at[b, :, pl.ds(r0, 16), :], stg)
            pltpu.sync_copy(fwp_hbm.at[b, :, pl.ds(r0, 16), :], stgf)

            def zero_body(i, _):
                acc[pl.ds(i * 16, 16)] = jnp.zeros((16,), jnp.float32)
                return 0

            lax.fori_loop(0, CHUNK // 16, zero_body, 0)

            slab = hh * 16 + ch
            rmin = bv[slab, pl.ds(0, 16)][0]
            rmax = bv[32 + slab, pl.ds(0, 16)][0]

            for bd in range(NBANDS):
                blo = bd * BAND_ROWS
                bhi = min((bd + 1) * BAND_ROWS, H)
                rn = min(RES_ROWS, H - blo)
                sz = rn * W

                @pl.when((rmin < float(bhi)) & (rmax >= float(blo)))
                def _():
                    pltpu.sync_copy(s2_hbm.at[b, pl.ds(blo * 4, sz // 128), :],
                                    band.at[pl.ds(0, sz // 128), :])

                    def px_body(i, _):
                        r = i // 32
                        c16 = (i % 32) * 16
                        ia = stg[0, r, pl.ds(c16, 16)]
                        ib = stg[1, r, pl.ds(c16, 16)]
                        ic = stg[2, r, pl.ds(c16, 16)]
                        idd = stg[3, r, pl.ds(c16, 16)]
                        fwa = stgf[0, r, pl.ds(c16, 16)]
                        fwb = stgf[1, r, pl.ds(c16, 16)]
                        fwc = stgf[2, r, pl.ds(c16, 16)]
                        fwd = stgf[3, r, pl.ds(c16, 16)]
                        row = ia >> 9
                        msk = (row >= blo) & (row < bhi)
                        base = blo * W
                        tot = jnp.zeros((16,), jnp.float32)
                        for idxv, fwv in ((ia, fwa), (ib, fwb), (ic, fwc), (idd, fwd)):
                            rel = jnp.clip(idxv - base, 0, sz - 1)
                            val = plsc.load_gather(band, [rel >> 7, rel & 127])
                            tot = tot + fwv * val
                        sel = jnp.where(msk, tot, 0.0)
                        plsc.addupdate(acc.at[pl.ds(i * 16, 16)], sel)
                        return 0

                    lax.fori_loop(0, CHUNK // 16, px_body, 0)

            pltpu.sync_copy(acc, out_hbm.at[b, pl.ds(hh * (HW // 2) + ch * CHUNK, CHUNK)])
            return 0

        lax.fori_loop(0, 16, do_chunk, 0)

    return k


def _sc_compact_kernel():
    mesh = plsc.VectorSubcoreMesh(core_axis_name="c", subcore_axis_name="s")
    info = plsc.get_sparse_core_info()
    nc = info.num_cores
    nchunks = 16
    csz = HW // nchunks

    @functools.partial(
        pl.kernel,
        mesh=mesh,
        out_type=(
            jax.ShapeDtypeStruct((32, CAP), jnp.float32),
            jax.ShapeDtypeStruct((32, CAP), jnp.int32),
            jax.ShapeDtypeStruct((32, 128), jnp.int32),
        ),
        compiler_params=pltpu.CompilerParams(needs_layout_passes=False),
        scratch_types=[
            pltpu.VMEM((csz,), jnp.float32),
            pltpu.VMEM((128,), jnp.float32),
            pltpu.VMEM((CAP,), jnp.float32),
            pltpu.VMEM((CAP,), jnp.int32),
            pltpu.VMEM((16,), jnp.int32),
        ],
    )
    def k(nms_hbm, t_hbm, cv_hbm, ci_hbm, cnt_hbm, chunk, tv, vbuf, ibuf, cbuf):
        wid = lax.axis_index("s") * nc + lax.axis_index("c")
        pltpu.sync_copy(t_hbm.at[wid], tv)
        t = tv[pl.ds(0, 16)][0]
        lane = lax.iota(jnp.int32, 16)

        def do_chunk(ch, off):
            pltpu.sync_copy(nms_hbm.at[wid, pl.ds(ch * csz, csz)], chunk)

            def body(i, off):
                v = chunk[pl.ds(i * 16, 16)]
                m = v >= t
                cnt = plsc.all_reduce_population_count(m)[0]

                @pl.when((cnt > 0) & (off <= CAP - 16))
                def _():
                    plsc.store_compressed(vbuf.at[pl.ds(off, 16)], v, mask=m)
                    iv = ch * csz + i * 16 + lane
                    plsc.store_compressed(ibuf.at[pl.ds(off, 16)], iv, mask=m)

                return jnp.where(off <= CAP - 16, off + cnt, off)

            return lax.fori_loop(0, csz // 16, body, off)

        off = lax.fori_loop(0, nchunks, do_chunk, jnp.int32(0))
        cbuf[...] = jnp.full((16,), jnp.int32(0)) + off
        pltpu.sync_copy(vbuf, cv_hbm.at[wid])
        pltpu.sync_copy(ibuf, ci_hbm.at[wid])
        pltpu.sync_copy(cbuf, cnt_hbm.at[wid, pl.ds(0, 16)])

    return k


def _select_compact(v, iv, count):
    lanes = lax.broadcasted_iota(jnp.int32, (1, CAP), 1)
    validm = lanes < count
    bits = lax.bitcast_convert_type(v, jnp.int32)
    mbits = jnp.where(validm, bits, -1)

    def bs_body(_, carry):
        lo, hi = carry
        mid = lo + (hi - lo) // 2
        cnt = jnp.sum((mbits >= mid).astype(jnp.float32))
        big = cnt >= float(TOP_K)
        return (jnp.where(big, mid, lo), jnp.where(big, hi, mid))

    lo, _ = lax.fori_loop(0, 32, bs_body, (jnp.int32(-1), jnp.int32(0x7F800001)))

    gt = (mbits > lo).astype(jnp.float32)
    eq = (mbits == lo).astype(jnp.float32)
    n_gt = jnp.sum(gt)
    need_eq = float(TOP_K) - n_gt

    iu = lax.broadcasted_iota(jnp.int32, (128, 128), 0).astype(jnp.float32)
    ju = lax.broadcasted_iota(jnp.int32, (128, 128), 1).astype(jnp.float32)
    uex = (iu < ju).astype(jnp.float32)
    i8 = lax.broadcasted_iota(jnp.int32, (8, 8), 0).astype(jnp.float32)
    j8 = lax.broadcasted_iota(jnp.int32, (8, 8), 1).astype(jnp.float32)
    l8 = (j8 < i8).astype(jnp.float32)

    def excl_cumsum(x):
        x8 = jnp.reshape(x, (8, 128))
        p = lax.dot_general(x8, uex, (((1,), (0,)), ((), ())),
                            preferred_element_type=jnp.float32,
                         precision=lax.Precision.HIGHEST)
        rs = p[:, 127:128] + x8[:, 127:128]
        er = lax.dot_general(l8, rs, (((1,), (0,)), ((), ())),
                             preferred_element_type=jnp.float32,
                         precision=lax.Precision.HIGHEST)
        return jnp.reshape(p + er, (1, CAP))

    eqrank = excl_cumsum(eq)
    sel = gt + eq * (eqrank < need_eq).astype(jnp.float32)
    r0 = excl_cumsum(sel)

    iota_r = lax.broadcasted_iota(jnp.int32, (TOP_K, 1), 0).astype(jnp.float32)
    ot = (r0 == iota_r).astype(jnp.float32) * sel
    ivf = iv.astype(jnp.float32)
    vc = lax.dot_general(ot, v, (((1,), (1,)), ((), ())),
                         preferred_element_type=jnp.float32,
                         precision=lax.Precision.HIGHEST)
    ic = lax.dot_general(ot, ivf, (((1,), (1,)), ((), ())),
                         preferred_element_type=jnp.float32,
                         precision=lax.Precision.HIGHEST)
    return vc, ic


def _gmat(coordT, scaleT):
    iot = lax.broadcasted_iota(jnp.int32, (TOP_K, TOP_K), 0).astype(jnp.float32)
    d = iot - coordT
    ad = jnp.abs(d)
    g = ((ad == 0.0).astype(jnp.float32) * _G0
         + (ad == 1.0).astype(jnp.float32) * _G1
         + (ad == 2.0).astype(jnp.float32) * _G2)
    return g * scaleT


def _final_body(cv1_ref, ci1_ref, cv2_ref, ci2_ref, cnt_ref, vis_ref,
                kp_ref, sq_ref, vs_ref):
    b = pl.program_id(0)
    c1 = cnt_ref[b, 0]
    c2 = cnt_ref[b + 16, 0]

    vc1, ic1 = _select_compact(cv1_ref[0], ci1_ref[0], c1)
    vc2, ic2 = _select_compact(cv2_ref[0], ci2_ref[0], c2)

    def split_xy(ic):
        yc = jnp.floor(ic * (1.0 / W))
        xc = ic - yc * float(W)
        return xc, yc

    xc1, yc1 = split_xy(ic1)
    xc2, yc2 = split_xy(ic2)

    vT = jnp.transpose(vc1)
    iT = jnp.transpose(ic1)
    rank = jnp.sum((vT > vc1).astype(jnp.float32)
                   + (vT == vc1).astype(jnp.float32) * (iT < ic1).astype(jnp.float32),
                   axis=1, keepdims=True)
    rT = jnp.transpose(rank)
    iota_r = lax.broadcasted_iota(jnp.int32, (TOP_K, 1), 0).astype(jnp.float32)
    perm = (rT == iota_r).astype(jnp.float32)
    kpx = lax.dot_general(perm, xc1, (((1,), (0,)), ((), ())),
                          preferred_element_type=jnp.float32,
                         precision=lax.Precision.HIGHEST)
    kpy = lax.dot_general(perm, yc1, (((1,), (0,)), ((), ())),
                          preferred_element_type=jnp.float32,
                         precision=lax.Precision.HIGHEST)
    kp = jnp.concatenate([kpx, kpy], axis=1).astype(jnp.int32)
    kp_ref[...] = kp[None]

    u1 = _gmat(jnp.transpose(yc1), jnp.transpose(vc1) * _INVZ)
    v1 = _gmat(jnp.transpose(xc1), jnp.ones((1, TOP_K), jnp.float32))
    u2 = _gmat(jnp.transpose(yc2), jnp.transpose(vc2) * _INVZ)
    v2 = _gmat(jnp.transpose(xc2), jnp.ones((1, TOP_K), jnp.float32))

    d1 = lax.dot_general(u1, v1, (((1,), (1,)), ((), ())),
                         preferred_element_type=jnp.float32,
                         precision=lax.Precision.HIGHEST)
    d2 = lax.dot_general(u2, v2, (((1,), (1,)), ((), ())),
                         preferred_element_type=jnp.float32,
                         precision=lax.Precision.HIGHEST)
    diff = d1 - d2

    vis = vis_ref[0, 0]
    sqp = jnp.sum(diff * diff * vis)
    vsp = jnp.sum(vis)

    @pl.when(b == 0)
    def _():
        sq_ref[...] = jnp.zeros_like(sq_ref)
        vs_ref[...] = jnp.zeros_like(vs_ref)

    rows = lax.broadcasted_iota(jnp.int32, sq_ref.shape, 0)
    sq_ref[...] += jnp.where(rows == b, sqp, 0.0)
    vs_ref[...] += jnp.where(rows == b, vsp, 0.0)


def _final(cand_v, cand_i, counts, vis):
    cv3 = cand_v.reshape(32, 1, CAP)
    ci3 = cand_i.reshape(32, 1, CAP)
    return pl.pallas_call(
        _final_body,
        grid=(B,),
        in_specs=[
            pl.BlockSpec((1, 1, CAP), lambda b: (b, 0, 0)),
            pl.BlockSpec((1, 1, CAP), lambda b: (b, 0, 0)),
            pl.BlockSpec((1, 1, CAP), lambda b: (b + 16, 0, 0)),
            pl.BlockSpec((1, 1, CAP), lambda b: (b + 16, 0, 0)),
            pl.BlockSpec((32, 128), lambda b: (0, 0)),
            pl.BlockSpec((1, 1, H, W), lambda b: (b, 0, 0, 0)),
        ],
        out_specs=[
            pl.BlockSpec((1, TOP_K, 2), lambda b: (b, 0, 0)),
            pl.BlockSpec((B, 128), lambda b: (0, 0)),
            pl.BlockSpec((B, 128), lambda b: (0, 0)),
        ],
        out_shape=[
            jax.ShapeDtypeStruct((B, TOP_K, 2), jnp.int32),
            jax.ShapeDtypeStruct((B, 128), jnp.float32),
            jax.ShapeDtypeStruct((B, 128), jnp.float32),
        ],
    )(cv3, ci3, cv3, ci3, counts, vis)


def kernel(score1, score2, homo12):
    ys, xs = jnp.meshgrid(jnp.arange(H, dtype=jnp.float32),
                          jnp.arange(W, dtype=jnp.float32), indexing='ij')
    grid = jnp.stack([xs.reshape(-1), ys.reshape(-1),
                      jnp.ones(H * W, dtype=jnp.float32)], axis=0)
    warped = homo12 @ grid[None]
    warped = warped / (warped[:, 2:3, :] + 1e-8)
    xw = warped[:, 0, :].reshape(B, 1, H, W)
    yw = warped[:, 1, :].reshape(B, 1, H, W)

    planes, fwp, vis_mask1, nms1, t1, bounds = _warp_prep(xw, yw, score1)

    s2flat = score2.reshape(B, HW // 128, 128)
    w_flat = _sc_gather_kernel()(planes, fwp, s2flat, bounds)
    w_score2 = w_flat.reshape(B, 1, H, W)

    nms2, t2 = _nms2(w_score2)

    nms_all = jnp.concatenate([nms1.reshape(B, HW), nms2.reshape(B, HW)], axis=0)
    t_all = jnp.concatenate([t1, t2], axis=0)
    cand_v, cand_i, counts = _sc_compact_kernel()(nms_all, t_all)

    kp1, sq, vs = _final(cand_v, cand_i, counts, vis_mask1)

    loss = (jnp.sum(sq[:, 0]) / jnp.sum(vs[:, 0])) * LOSS_LAMBDA
    return (loss, kp1, vis_mask1)
